# baseline (speedup 1.0000x reference)
import functools
import jax
import jax.numpy as jnp
from jax import lax
from jax.experimental import pallas as pl
from jax.experimental.pallas import tpu as pltpu
from jax.experimental.pallas import tpu_sc as plsc

D = 64
SCALE = 8.0
L = 16
NBUF = 4
LEAD = 2
BLK = 128


def _make_emb_kernel(n_rows: int, row_len: int, num_cores: int):
    n_tc = n_rows // BLK
    mesh = plsc.VectorSubcoreMesh(core_axis_name="c", subcore_axis_name="s")

    @functools.partial(
        pl.kernel,
        out_type=jax.ShapeDtypeStruct((row_len, D // 8, n_tc, 8, BLK),
                                      jnp.float32),
        mesh=mesh,
        scratch_types=[
            pltpu.VMEM((BLK, row_len), jnp.int32),
            pltpu.VMEM((NBUF, BLK), jnp.int32),
            pltpu.VMEM((NBUF, BLK), jnp.int32),
            pltpu.VMEM((NBUF, BLK, BLK), jnp.float32),
            pltpu.VMEM((NBUF, D // 8, 8, BLK), jnp.float32),
            pltpu.SemaphoreType.DMA((NBUF,)),
            pltpu.SemaphoreType.DMA((NBUF,)),
        ],
        compiler_params=pltpu.CompilerParams(use_tc_tiling_on_sc=False,
                                             needs_layout_passes=False),
    )
    def _emb(x_hbm, t2_hbm, out_hbm, xv, qv, hv, pair, obuf, gsem, osem):
        wid = lax.axis_index("s") * num_cores + lax.axis_index("c")
        pltpu.sync_copy(x_hbm.at[pl.ds(wid * BLK, BLK)], xv)
        iota = lax.iota(jnp.int32, L)

        def prep(j, b):
            for g in range(BLK // L):
                rowv = iota + (g * L)
                tok = plsc.load_gather(xv, [rowv, jnp.full((L,), j, jnp.int32)])
                qv[b, pl.ds(g * L, L)] = lax.shift_right_logical(tok, 1)
                hv[b, pl.ds(g * L, L)] = (tok & 1) * D

        def g_desc(b):
            return (t2_hbm.at[qv.at[b]], pair.at[b], gsem.at[b])

        def start_gather(b):
            pltpu.async_copy(*g_desc(b))

        def wait_gather(b):
            pltpu.make_async_copy(*g_desc(b)).wait()

        def o_desc(j, b):
            return (obuf.at[b], out_hbm.at[j, :, wid], osem.at[b])

        def start_out(j, b):
            pltpu.async_copy(*o_desc(j, b))

        def wait_out(j, b):
            pltpu.make_async_copy(*o_desc(j, b)).wait()

        def transpose(b):
            hvecs = tuple(hv[b, pl.ds(g * L, L)] for g in range(BLK // L))
            rvecs = tuple(iota + (g * L) for g in range(BLK // L))

            def k_body(k, carry):
                hs, rs = carry
                tr = lax.shift_right_logical(k, 3)
                k8 = k & 7
                for g in range(BLK // L):
                    colv = hs[g] + k
                    v = plsc.load_gather(pair.at[b], [rs[g], colv])
                    obuf[b, tr, k8, pl.ds(g * L, L)] = v * SCALE
                return carry

            lax.fori_loop(0, D, k_body, (hvecs, rvecs))

        def step(j, b, *, first=False, last=False):
            wait_gather(b)
            if not first:
                wait_out(j - NBUF, b)
            transpose(b)
            start_out(j, b)
            if not last:
                nb = (b + LEAD) % NBUF
                prep(j + LEAD, nb)
                start_gather(nb)

        for j in range(LEAD):
            prep(j, j)
            start_gather(j)

        for b in range(NBUF):
            step(b, b, first=True)

        def group_body(g, carry):
            for b in range(NBUF):
                step(g * NBUF + b, b)
            return carry

        lax.fori_loop(1, row_len // NBUF - 1, group_body, 0)

        j0 = row_len - NBUF
        for b in range(NBUF):
            step(j0 + b, b, last=(j0 + b + LEAD >= row_len))

        for b in range(NBUF):
            wait_out(j0 + b, b)

    return _emb


@jax.jit
def _kernel_impl(x, table):
    info = plsc.get_sparse_core_info()
    n_rows, row_len = x.shape
    vocab = table.shape[0]
    t2 = jnp.reshape(table, (vocab // 2, 2 * D))
    emb = _make_emb_kernel(n_rows, row_len, info.num_cores)
    p5 = emb(x.astype(jnp.int32), t2)
    t = jnp.transpose(p5, (2, 4, 0, 1, 3))
    return jnp.reshape(t, (n_rows, row_len, D))


_DEBUG_ONCE = []


def _debug_report(x, table):
    if _DEBUG_ONCE:
        return
    _DEBUG_ONCE.append(1)
    import sys
    import re
    try:
        hlo = jax.jit(_kernel_impl.__wrapped__).lower(x, table).compile().as_text()
        for line in hlo.splitlines():
            if re.search(r"bitcast\(|copy\(|reshape\(|ROOT|f32\[500000", line):
                print("DBG:", line.strip()[:170], file=sys.stderr)
    except Exception as e:
        print("DBG fail:", repr(e), file=sys.stderr)


def kernel(x, table):
    _debug_report(x, table)
    return _kernel_impl(x, table)
